# baseline (speedup 1.0000x reference)
import functools

import jax
import jax.numpy as jnp
from jax import lax
from jax.experimental import pallas as pl
from jax.experimental.pallas import tpu as pltpu
from jax.experimental.pallas import tpu_sc as plsc

RING = 8


def _sc_geometry():
    try:
        info = plsc.get_sparse_core_info()
        return info.num_cores, info.num_subcores
    except Exception:
        return 2, 16


@functools.lru_cache(maxsize=None)
def _build(B, V, D, nc, ns):
    nw = nc * ns
    b_per_w = B // nw
    n_groups = b_per_w // RING
    mesh = plsc.VectorSubcoreMesh(core_axis_name="c", subcore_axis_name="s")

    @functools.partial(
        pl.kernel,
        mesh=mesh,
        out_type=jax.ShapeDtypeStruct((D, B), jnp.float32),
        scratch_types=[
            pltpu.VMEM((b_per_w + 16,), jnp.int32),
            pltpu.VMEM((RING, D, 128), jnp.float32),
            pltpu.VMEM((D, b_per_w), jnp.float32),
        ]
        + [pltpu.SemaphoreType.DMA] * RING,
        compiler_params=pltpu.CompilerParams(
            use_tc_tiling_on_sc=True, needs_layout_passes=False
        ),
    )
    def k(idx_hbm, tab_t_hbm, out_t_hbm, idx_vm, ring_v, osg_v, *sems):
        wid = lax.axis_index("s") * nc + lax.axis_index("c")
        base = wid * b_per_w
        pltpu.sync_copy(idx_hbm.at[pl.ds(base, b_per_w)], idx_vm.at[pl.ds(0, b_per_w)])

        rows16 = [lax.iota(jnp.int32, 16) + 16 * t for t in range(D // 16)]

        def fetch(i, r):
            col = pl.multiple_of((i >> 7) << 7, 128)
            pltpu.async_copy(
                tab_t_hbm.at[:, pl.ds(col, 128)], ring_v.at[r], sems[r]
            )

        v0 = idx_vm[pl.ds(0, 16)]
        for r in range(RING):
            fetch(v0[r], r)

        def body(o, _):
            v = idx_vm[pl.ds(o * RING, 16)]
            for r in range(RING):
                pltpu.make_async_copy(
                    tab_t_hbm.at[:, pl.ds(0, 128)], ring_v.at[r], sems[r]
                ).wait()
                lane = jnp.full((16,), v[r] & 127, jnp.int32)
                kk = jnp.full((16,), o * RING + r, jnp.int32)
                for t in range(D // 16):
                    g = plsc.load_gather(ring_v.at[r], [rows16[t], lane])
                    plsc.store_scatter(osg_v, [rows16[t], kk], g)

                @pl.when(o < n_groups - 1)
                def _():
                    fetch(v[RING + r], r)

            return 0

        lax.fori_loop(0, n_groups, body, 0)
        pltpu.sync_copy(osg_v, out_t_hbm.at[:, pl.ds(base, b_per_w)])

    return k


def kernel(indices, table):
    (B,) = indices.shape
    V, D = table.shape
    nc, ns = _sc_geometry()
    out_t = _build(B, V, D, nc, ns)(indices.astype(jnp.int32), table.T)
    return out_t.T
